# baseline (speedup 1.0000x reference)
import functools

import jax
import jax.numpy as jnp
from jax import lax
from jax.experimental import pallas as pl
from jax.experimental.pallas import tpu as pltpu
from jax.experimental.pallas import tpu_sc as plsc

_N = 10000
_E = 320000
_DIN = 128
_DH = 128
_DH2 = 256
_DOUT = 32
_NSEED = 1024

_NC = 2
_NS = 16
_NW = _NC * _NS
_SB = 128
_CH = ((-(-_E // (_NW * _SB)) + 1) // 2) * 2
_EP = _NW * _SB * _CH
_NP = 10240
_RPT = _NP // _NS
_DEGW = 16
_HW = 64

_BR = 2000


def _mesh():
    return plsc.VectorSubcoreMesh(
        core_axis_name="c", subcore_axis_name="s",
        num_cores=_NC, num_subcores=_NS)


def _fill_rows(ref, nrows, ncol, value):
    v = jnp.full((16,), value, jnp.float32)

    def body(r, carry):
        for k in range(ncol // 16):
            ref[r, pl.ds(k * 16, 16)] = v
        return carry

    lax.fori_loop(0, nrows, body, 0)


def _segsum_body(feat, with_deg, *refs):
    if with_deg:
        (x_hbm, srcw, dstw, agg_hbm, deg_hbm,
         idx_v, dst_v, rows, rows2, ones_v, zdeg_v, acc_sh, accd_sh,
         sem, sem2, ssem, dsem) = refs
    else:
        (x_hbm, srcw, dstw, agg_hbm,
         idx_v, dst_v, rows, rows2, acc_sh, sem, sem2, ssem) = refs
        dsem = None
    c = lax.axis_index("c")
    s = lax.axis_index("s")
    wid = c * _NS + s
    base = s * _RPT

    _fill_rows(rows, _SB, feat, 0.0)
    if with_deg:
        _fill_rows(ones_v, _SB, _DEGW, 1.0)
        _fill_rows(zdeg_v, _SB, _DEGW, 0.0)
    pltpu.sync_copy(srcw.at[wid], idx_v)
    pltpu.sync_copy(dstw.at[wid], dst_v)
    for k in range(_RPT // _SB):
        pltpu.sync_copy(rows, acc_sh.at[pl.ds(base + k * _SB, _SB)])
        if with_deg:
            pltpu.sync_copy(zdeg_v, accd_sh.at[pl.ds(base + k * _SB, _SB)])
    plsc.subcore_barrier()

    def gissue(j, buf, gsem):
        pltpu.async_copy(x_hbm.at[idx_v.at[j]], buf, gsem)

    def gwait(buf, gsem):
        pltpu.make_async_copy(x_hbm.at[idx_v.at[0]], buf, gsem).wait()

    def visit(j, buf, gsem, nxt):
        if nxt is not None:
            gissue(j + 1, *nxt)
        gwait(buf, gsem)
        pltpu.async_copy(buf, acc_sh.at[dst_v.at[j]], ssem, add=True)
        if with_deg:
            pltpu.async_copy(ones_v, accd_sh.at[dst_v.at[j]], dsem, add=True)

    gissue(0, rows, sem)

    def body(i, carry):
        j = 2 * i
        visit(j, rows, sem, (rows2, sem2))
        visit(j + 1, rows2, sem2, (rows, sem))
        return carry

    lax.fori_loop(0, _CH // 2 - 1, body, 0)
    visit(_CH - 2, rows, sem, (rows2, sem2))
    visit(_CH - 1, rows2, sem2, None)

    def sdrain(i, carry):
        pltpu.make_async_copy(rows, acc_sh.at[dst_v.at[0]], ssem).wait()
        if with_deg:
            pltpu.make_async_copy(
                ones_v, accd_sh.at[dst_v.at[0]], dsem).wait()
        return carry

    lax.fori_loop(0, _CH, sdrain, 0)
    plsc.subcore_barrier()
    pltpu.sync_copy(acc_sh.at[pl.ds(base, _RPT)],
                    agg_hbm.at[c, pl.ds(base, _RPT)])
    if with_deg:
        pltpu.sync_copy(accd_sh.at[pl.ds(base, _RPT)],
                        deg_hbm.at[c, pl.ds(base, _RPT)])


def _make_segsum(feat, with_deg):
    out_type = [jax.ShapeDtypeStruct((_NC, _NP, feat), jnp.float32)]
    scratch = [
        pltpu.VMEM((_CH, _SB), jnp.int32),
        pltpu.VMEM((_CH, _SB), jnp.int32),
        pltpu.VMEM((_SB, feat), jnp.float32),
        pltpu.VMEM((_SB, feat), jnp.float32),
    ]
    if with_deg:
        out_type.append(jax.ShapeDtypeStruct((_NC, _NP, _DEGW), jnp.float32))
        scratch += [
            pltpu.VMEM((_SB, _DEGW), jnp.float32),
            pltpu.VMEM((_SB, _DEGW), jnp.float32),
        ]
    scratch.append(pltpu.VMEM_SHARED((_NP, feat), jnp.float32))
    if with_deg:
        scratch.append(pltpu.VMEM_SHARED((_NP, _DEGW), jnp.float32))
    scratch += [pltpu.SemaphoreType.DMA] * (4 if with_deg else 3)
    return pl.kernel(
        functools.partial(_segsum_body, feat, with_deg),
        out_type=out_type if with_deg else out_type[0],
        mesh=_mesh(),
        scratch_types=scratch,
        compiler_params=pltpu.CompilerParams(use_tc_tiling_on_sc=False),
    )


def _segsum_gate_body(p_hbm, srcw, dstw, etw, wt16, bt16, agg_hbm,
                      idx_v, dst_v, et_v, rows, rows2, wt_v, bt_v,
                      acc_sh, sem, sem2, ssem):
    c = lax.axis_index("c")
    s = lax.axis_index("s")
    wid = c * _NS + s
    base = s * _RPT

    _fill_rows(rows, _SB, _DOUT, 0.0)
    pltpu.sync_copy(srcw.at[wid], idx_v)
    pltpu.sync_copy(dstw.at[wid], dst_v)
    pltpu.sync_copy(etw.at[wid], et_v)
    pltpu.sync_copy(wt16, wt_v)
    pltpu.sync_copy(bt16, bt_v)
    for k in range(_RPT // _SB):
        pltpu.sync_copy(rows, acc_sh.at[pl.ds(base + k * _SB, _SB)])
    plsc.subcore_barrier()

    def gissue(j, buf, gsem):
        pltpu.async_copy(p_hbm.at[idx_v.at[j]], buf, gsem)

    def gwait(buf, gsem):
        pltpu.make_async_copy(p_hbm.at[idx_v.at[0]], buf, gsem).wait()

    def visit(j, buf, gsem, nxt):
        if nxt is not None:
            gissue(j + 1, *nxt)
        gwait(buf, gsem)
        wv = wt_v[...]
        bv = bt_v[...]
        for g in range(_SB // 16):
            z = et_v[j, pl.ds(g * 16, 16)]
            gval = 1.0 / (1.0 + jnp.exp(-(z * wv + bv)))
            for l in range(16):
                e = g * 16 + l
                gvec = jnp.full((16,), gval[l], jnp.float32)
                for k in range(_DOUT // 16):
                    sl = pl.ds(k * 16, 16)
                    buf[e, sl] = buf[e, sl] * gvec
        pltpu.async_copy(buf, acc_sh.at[dst_v.at[j]], ssem, add=True)

    gissue(0, rows, sem)

    def body(i, carry):
        j = 2 * i
        visit(j, rows, sem, (rows2, sem2))
        visit(j + 1, rows2, sem2, (rows, sem))
        return carry

    lax.fori_loop(0, _CH // 2 - 1, body, 0)
    visit(_CH - 2, rows, sem, (rows2, sem2))
    visit(_CH - 1, rows2, sem2, None)

    def sdrain(i, carry):
        pltpu.make_async_copy(rows, acc_sh.at[dst_v.at[0]], ssem).wait()
        return carry

    lax.fori_loop(0, _CH, sdrain, 0)
    plsc.subcore_barrier()
    pltpu.sync_copy(acc_sh.at[pl.ds(base, _RPT)],
                    agg_hbm.at[c, pl.ds(base, _RPT)])


_segsum_gate = pl.kernel(
    _segsum_gate_body,
    out_type=jax.ShapeDtypeStruct((_NC, _NP, _DOUT), jnp.float32),
    mesh=_mesh(),
    scratch_types=[
        pltpu.VMEM((_CH, _SB), jnp.int32),
        pltpu.VMEM((_CH, _SB), jnp.int32),
        pltpu.VMEM((_CH, _SB), jnp.float32),
        pltpu.VMEM((_SB, _DOUT), jnp.float32),
        pltpu.VMEM((_SB, _DOUT), jnp.float32),
        pltpu.VMEM((16,), jnp.float32),
        pltpu.VMEM((16,), jnp.float32),
        pltpu.VMEM_SHARED((_NP, _DOUT), jnp.float32),
        pltpu.SemaphoreType.DMA,
        pltpu.SemaphoreType.DMA,
        pltpu.SemaphoreType.DMA,
    ],
    compiler_params=pltpu.CompilerParams(use_tc_tiling_on_sc=False),
)


def _final_body(a0_hbm, a1_hbm, d0_hbm, d1_hbm, b3_hbm, seedw, out_hbm,
                sidx_v, a0, a1, d0, d1, b3_v, outb, sem):
    c = lax.axis_index("c")
    s = lax.axis_index("s")
    wid = c * _NS + s
    spw = _NSEED // _NW

    pltpu.sync_copy(seedw.at[wid], sidx_v)
    pltpu.sync_copy(b3_hbm, b3_v)
    pltpu.async_copy(a0_hbm.at[sidx_v], a0, sem).wait()
    pltpu.async_copy(a1_hbm.at[sidx_v], a1, sem).wait()
    pltpu.async_copy(d0_hbm.at[sidx_v], d0, sem).wait()
    pltpu.async_copy(d1_hbm.at[sidx_v], d1, sem).wait()

    for r in range(spw):
        dvec = d0[r, pl.ds(0, 16)] + d1[r, pl.ds(0, 16)]
        invv = 1.0 / jnp.maximum(dvec, 1.0)
        iv = jnp.full((16,), invv[0], jnp.float32)
        for k in range(_DOUT // 16):
            sl = pl.ds(k * 16, 16)
            outb[r, sl] = (a0[r, sl] + a1[r, sl]) * iv + b3_v[sl]

    pltpu.sync_copy(outb, out_hbm.at[pl.ds(wid * spw, spw)])


_final = pl.kernel(
    _final_body,
    out_type=jax.ShapeDtypeStruct((_NSEED, _DOUT), jnp.float32),
    mesh=_mesh(),
    scratch_types=[
        pltpu.VMEM((_NSEED // _NW,), jnp.int32),
        pltpu.VMEM((_NSEED // _NW, _DOUT), jnp.float32),
        pltpu.VMEM((_NSEED // _NW, _DOUT), jnp.float32),
        pltpu.VMEM((_NSEED // _NW, _DEGW), jnp.float32),
        pltpu.VMEM((_NSEED // _NW, _DEGW), jnp.float32),
        pltpu.VMEM((_DOUT,), jnp.float32),
        pltpu.VMEM((_NSEED // _NW, _DOUT), jnp.float32),
        pltpu.SemaphoreType.DMA,
    ],
    compiler_params=pltpu.CompilerParams(use_tc_tiling_on_sc=False),
)


def _inv_deg(deg_ref):
    deg = deg_ref[0, :, 0:1] + deg_ref[1, :, 0:1]
    return 1.0 / jnp.maximum(deg, 1.0)


def _layer1_tc(x_ref, agg0_ref, agg1_ref, deg_ref, w1r_ref, w1n0_ref,
               w1n1_ref, b1_ref, w2n_ref, w3_ref, h1a_ref, h1b_ref, q_ref):
    inv = _inv_deg(deg_ref)
    a0 = (agg0_ref[0] + agg0_ref[1]) * inv
    a1 = (agg1_ref[0] + agg1_ref[1]) * inv
    h = jnp.dot(x_ref[...], w1r_ref[...], preferred_element_type=jnp.float32)
    h = h + jnp.dot(a0, w1n0_ref[...], preferred_element_type=jnp.float32)
    h = h + jnp.dot(a1, w1n1_ref[...], preferred_element_type=jnp.float32)
    h = jax.nn.gelu(h + b1_ref[...][None, :])
    h1a_ref[...] = h[:, :_HW]
    h1b_ref[...] = h[:, _HW:]
    v = jnp.dot(w2n_ref[...], w3_ref[...], preferred_element_type=jnp.float32)
    q_ref[...] = jnp.dot(h, v, preferred_element_type=jnp.float32)


def _layer2_tc(h1a_ref, h1b_ref, aggq_ref, deg_ref, w2r_ref, b2_ref,
               w3_ref, p_ref):
    inv = _inv_deg(deg_ref)
    aq = (aggq_ref[0] + aggq_ref[1]) * inv
    u = jnp.dot(w2r_ref[...], w3_ref[...], preferred_element_type=jnp.float32)
    p = jnp.dot(h1a_ref[...], u[:_HW], preferred_element_type=jnp.float32)
    p = p + jnp.dot(h1b_ref[...], u[_HW:],
                    preferred_element_type=jnp.float32)
    c = jnp.dot(b2_ref[...][None, :], w3_ref[...],
                preferred_element_type=jnp.float32)
    p_ref[...] = p + aq + c


def _full(shape):
    nd = len(shape)
    return pl.BlockSpec(shape, lambda i: (0,) * nd)


_layer1 = pl.pallas_call(
    _layer1_tc,
    grid=(_N // _BR,),
    in_specs=[
        pl.BlockSpec((_BR, _DIN), lambda i: (i, 0)),
        pl.BlockSpec((_NC, _BR, _HW), lambda i: (0, i, 0)),
        pl.BlockSpec((_NC, _BR, _HW), lambda i: (0, i, 0)),
        pl.BlockSpec((_NC, _BR, _DEGW), lambda i: (0, i, 0)),
        _full((_DIN, _DH)),
        _full((_HW, _DH)),
        _full((_HW, _DH)),
        _full((_DH,)),
        _full((_DH, _DH2)),
        _full((_DH2, _DOUT)),
    ],
    out_specs=[
        pl.BlockSpec((_BR, _HW), lambda i: (i, 0)),
        pl.BlockSpec((_BR, _HW), lambda i: (i, 0)),
        pl.BlockSpec((_BR, _DOUT), lambda i: (i, 0)),
    ],
    out_shape=[
        jax.ShapeDtypeStruct((_N, _HW), jnp.float32),
        jax.ShapeDtypeStruct((_N, _HW), jnp.float32),
        jax.ShapeDtypeStruct((_N, _DOUT), jnp.float32),
    ],
)

_layer2 = pl.pallas_call(
    _layer2_tc,
    grid=(_N // _BR,),
    in_specs=[
        pl.BlockSpec((_BR, _HW), lambda i: (i, 0)),
        pl.BlockSpec((_BR, _HW), lambda i: (i, 0)),
        pl.BlockSpec((_NC, _BR, _DOUT), lambda i: (0, i, 0)),
        pl.BlockSpec((_NC, _BR, _DEGW), lambda i: (0, i, 0)),
        _full((_DH, _DH2)),
        _full((_DH2,)),
        _full((_DH2, _DOUT)),
    ],
    out_specs=pl.BlockSpec((_BR, _DOUT), lambda i: (i, 0)),
    out_shape=jax.ShapeDtypeStruct((_N, _DOUT), jnp.float32),
)


def kernel(x, edge_index, edge_time, seed_idx, W1r, W1n, b1, W2r, W2n, b2,
           wt, bt, W3, b3):
    src = edge_index[0]
    dst = edge_index[1]
    pad = _EP - _E
    srcw = jnp.concatenate(
        [src, jnp.zeros((pad,), jnp.int32)]).reshape(_NW, _CH, _SB)
    dstw = jnp.concatenate(
        [dst, jnp.full((pad,), _N, jnp.int32)]).reshape(_NW, _CH, _SB)
    etw = jnp.concatenate(
        [edge_time, jnp.zeros((pad,), jnp.float32)]).reshape(_NW, _CH, _SB)
    seedw = seed_idx.reshape(_NW, _NSEED // _NW)
    wt16 = jnp.broadcast_to(wt.astype(jnp.float32), (16,))
    bt16 = jnp.broadcast_to(bt.astype(jnp.float32), (16,))

    xa = x[:, :_HW]
    xb = x[:, _HW:]
    seg_deg = _make_segsum(_HW, True)
    seg = _make_segsum(_HW, False)

    aggx0, degp = seg_deg(xa, srcw, dstw)
    aggx1 = seg(xb, srcw, dstw)
    h1a, h1b, q = _layer1(x, aggx0, aggx1, degp, W1r, W1n[:_HW], W1n[_HW:],
                          b1, W2n, W3)
    aggq = _make_segsum(_DOUT, False)(q, srcw, dstw)
    p = _layer2(h1a, h1b, aggq, degp, W2r, b2, W3)
    agg3 = _segsum_gate(p, srcw, dstw, etw, wt16, bt16)
    out = _final(agg3[0], agg3[1], degp[0], degp[1], b3, seedw)
    return out


# speedup vs baseline: 1.1777x; 1.1777x over previous
import functools

import jax
import jax.numpy as jnp
from jax import lax
from jax.experimental import pallas as pl
from jax.experimental.pallas import tpu as pltpu
from jax.experimental.pallas import tpu_sc as plsc

_N = 10000
_E = 320000
_DIN = 128
_DH = 128
_DH2 = 256
_DOUT = 32
_NSEED = 1024

_NC = 2
_NS = 16
_NW = _NC * _NS
_SB = 128
_CH = -(-_E // (_NW * _SB))
_EP = _NW * _SB * _CH
_NP = 10032
_RPT = _NP // _NS
_DEGW = 16
_HW = 64

_BR = 2000


def _mesh():
    return plsc.VectorSubcoreMesh(
        core_axis_name="c", subcore_axis_name="s",
        num_cores=_NC, num_subcores=_NS)


def _fill_rows(ref, nrows, ncol, value):
    v = jnp.full((16,), value, jnp.float32)

    def body(r, carry):
        for k in range(ncol // 16):
            ref[r, pl.ds(k * 16, 16)] = v
        return carry

    lax.fori_loop(0, nrows, body, 0)


def _segsum_body(feat, with_deg, *refs):
    if with_deg:
        (x_hbm, srcw, dstw, agg_hbm, deg_hbm,
         idx_v, dst_v, rows, ones_v, zdeg_v, acc_sh, accd_sh, sem) = refs
    else:
        (x_hbm, srcw, dstw, agg_hbm,
         idx_v, dst_v, rows, acc_sh, sem) = refs
    c = lax.axis_index("c")
    s = lax.axis_index("s")
    wid = c * _NS + s
    base = s * _RPT

    _fill_rows(rows, _SB, feat, 0.0)
    if with_deg:
        _fill_rows(ones_v, _SB, _DEGW, 1.0)
        _fill_rows(zdeg_v, _SB, _DEGW, 0.0)
    pltpu.sync_copy(srcw.at[wid], idx_v)
    pltpu.sync_copy(dstw.at[wid], dst_v)
    off = 0
    while off < _RPT:
        n = min(_SB, _RPT - off)
        pltpu.sync_copy(rows.at[pl.ds(0, n)],
                        acc_sh.at[pl.ds(base + off, n)])
        if with_deg:
            pltpu.sync_copy(zdeg_v.at[pl.ds(0, n)],
                            accd_sh.at[pl.ds(base + off, n)])
        off += n
    plsc.subcore_barrier()

    def body(j, carry):
        pltpu.async_copy(x_hbm.at[idx_v.at[j]], rows, sem).wait()
        pltpu.sync_copy(rows, acc_sh.at[dst_v.at[j]], add=True)
        if with_deg:
            pltpu.sync_copy(ones_v, accd_sh.at[dst_v.at[j]], add=True)
        return carry

    lax.fori_loop(0, _CH, body, 0)
    plsc.subcore_barrier()
    pltpu.sync_copy(acc_sh.at[pl.ds(base, _RPT)],
                    agg_hbm.at[c, pl.ds(base, _RPT)])
    if with_deg:
        pltpu.sync_copy(accd_sh.at[pl.ds(base, _RPT)],
                        deg_hbm.at[c, pl.ds(base, _RPT)])


def _make_segsum(feat, with_deg):
    out_type = [jax.ShapeDtypeStruct((_NC, _NP, feat), jnp.float32)]
    scratch = [
        pltpu.VMEM((_CH, _SB), jnp.int32),
        pltpu.VMEM((_CH, _SB), jnp.int32),
        pltpu.VMEM((_SB, feat), jnp.float32),
    ]
    if with_deg:
        out_type.append(jax.ShapeDtypeStruct((_NC, _NP, _DEGW), jnp.float32))
        scratch += [
            pltpu.VMEM((_SB, _DEGW), jnp.float32),
            pltpu.VMEM((_SB, _DEGW), jnp.float32),
        ]
    scratch.append(pltpu.VMEM_SHARED((_NP, feat), jnp.float32))
    if with_deg:
        scratch.append(pltpu.VMEM_SHARED((_NP, _DEGW), jnp.float32))
    scratch.append(pltpu.SemaphoreType.DMA)
    return pl.kernel(
        functools.partial(_segsum_body, feat, with_deg),
        out_type=out_type if with_deg else out_type[0],
        mesh=_mesh(),
        scratch_types=scratch,
        compiler_params=pltpu.CompilerParams(use_tc_tiling_on_sc=False),
    )


def _segsum_gate_body(p_hbm, srcw, dstw, etw, wt16, bt16, agg_hbm,
                      idx_v, dst_v, et_v, rows, wt_v, bt_v,
                      acc_sh, sem):
    c = lax.axis_index("c")
    s = lax.axis_index("s")
    wid = c * _NS + s
    base = s * _RPT

    _fill_rows(rows, _SB, _DOUT, 0.0)
    pltpu.sync_copy(srcw.at[wid], idx_v)
    pltpu.sync_copy(dstw.at[wid], dst_v)
    pltpu.sync_copy(etw.at[wid], et_v)
    pltpu.sync_copy(wt16, wt_v)
    pltpu.sync_copy(bt16, bt_v)
    off = 0
    while off < _RPT:
        n = min(_SB, _RPT - off)
        pltpu.sync_copy(rows.at[pl.ds(0, n)],
                        acc_sh.at[pl.ds(base + off, n)])
        off += n
    plsc.subcore_barrier()

    def body(j, carry):
        pltpu.async_copy(p_hbm.at[idx_v.at[j]], rows, sem).wait()
        wv = wt_v[...]
        bv = bt_v[...]
        for g in range(_SB // 16):
            z = et_v[j, pl.ds(g * 16, 16)]
            gval = 1.0 / (1.0 + jnp.exp(-(z * wv + bv)))
            for l in range(16):
                e = g * 16 + l
                gvec = jnp.full((16,), gval[l], jnp.float32)
                for k in range(_DOUT // 16):
                    sl = pl.ds(k * 16, 16)
                    rows[e, sl] = rows[e, sl] * gvec
        pltpu.sync_copy(rows, acc_sh.at[dst_v.at[j]], add=True)
        return carry

    lax.fori_loop(0, _CH, body, 0)
    plsc.subcore_barrier()
    pltpu.sync_copy(acc_sh.at[pl.ds(base, _RPT)],
                    agg_hbm.at[c, pl.ds(base, _RPT)])


_segsum_gate = pl.kernel(
    _segsum_gate_body,
    out_type=jax.ShapeDtypeStruct((_NC, _NP, _DOUT), jnp.float32),
    mesh=_mesh(),
    scratch_types=[
        pltpu.VMEM((_CH, _SB), jnp.int32),
        pltpu.VMEM((_CH, _SB), jnp.int32),
        pltpu.VMEM((_CH, _SB), jnp.float32),
        pltpu.VMEM((_SB, _DOUT), jnp.float32),
        pltpu.VMEM((16,), jnp.float32),
        pltpu.VMEM((16,), jnp.float32),
        pltpu.VMEM_SHARED((_NP, _DOUT), jnp.float32),
        pltpu.SemaphoreType.DMA,
    ],
    compiler_params=pltpu.CompilerParams(use_tc_tiling_on_sc=False),
)


def _final_body(a0_hbm, a1_hbm, d0_hbm, d1_hbm, b3_hbm, seedw, out_hbm,
                sidx_v, a0, a1, d0, d1, b3_v, outb, sem):
    c = lax.axis_index("c")
    s = lax.axis_index("s")
    wid = c * _NS + s
    spw = _NSEED // _NW

    pltpu.sync_copy(seedw.at[wid], sidx_v)
    pltpu.sync_copy(b3_hbm, b3_v)
    pltpu.async_copy(a0_hbm.at[sidx_v], a0, sem).wait()
    pltpu.async_copy(a1_hbm.at[sidx_v], a1, sem).wait()
    pltpu.async_copy(d0_hbm.at[sidx_v], d0, sem).wait()
    pltpu.async_copy(d1_hbm.at[sidx_v], d1, sem).wait()

    for r in range(spw):
        dvec = d0[r, pl.ds(0, 16)] + d1[r, pl.ds(0, 16)]
        invv = 1.0 / jnp.maximum(dvec, 1.0)
        iv = jnp.full((16,), invv[0], jnp.float32)
        for k in range(_DOUT // 16):
            sl = pl.ds(k * 16, 16)
            outb[r, sl] = (a0[r, sl] + a1[r, sl]) * iv + b3_v[sl]

    pltpu.sync_copy(outb, out_hbm.at[pl.ds(wid * spw, spw)])


_final = pl.kernel(
    _final_body,
    out_type=jax.ShapeDtypeStruct((_NSEED, _DOUT), jnp.float32),
    mesh=_mesh(),
    scratch_types=[
        pltpu.VMEM((_NSEED // _NW,), jnp.int32),
        pltpu.VMEM((_NSEED // _NW, _DOUT), jnp.float32),
        pltpu.VMEM((_NSEED // _NW, _DOUT), jnp.float32),
        pltpu.VMEM((_NSEED // _NW, _DEGW), jnp.float32),
        pltpu.VMEM((_NSEED // _NW, _DEGW), jnp.float32),
        pltpu.VMEM((_DOUT,), jnp.float32),
        pltpu.VMEM((_NSEED // _NW, _DOUT), jnp.float32),
        pltpu.SemaphoreType.DMA,
    ],
    compiler_params=pltpu.CompilerParams(use_tc_tiling_on_sc=False),
)


def _inv_deg(deg_ref):
    deg = deg_ref[0, :, 0:1] + deg_ref[1, :, 0:1]
    return 1.0 / jnp.maximum(deg, 1.0)


def _layer1_tc(x_ref, agg_ref, deg_ref, w1r_ref, w1n_ref,
               b1_ref, w2n_ref, w3_ref, h1a_ref, h1b_ref, q_ref):
    inv = _inv_deg(deg_ref)
    a = (agg_ref[0] + agg_ref[1]) * inv
    h = jnp.dot(x_ref[...], w1r_ref[...], preferred_element_type=jnp.float32)
    h = h + jnp.dot(a, w1n_ref[...], preferred_element_type=jnp.float32)
    h = jax.nn.gelu(h + b1_ref[...][None, :])
    h1a_ref[...] = h[:, :_HW]
    h1b_ref[...] = h[:, _HW:]
    v = jnp.dot(w2n_ref[...], w3_ref[...], preferred_element_type=jnp.float32)
    q_ref[...] = jnp.dot(h, v, preferred_element_type=jnp.float32)


def _layer2_tc(h1a_ref, h1b_ref, aggq_ref, deg_ref, w2r_ref, b2_ref,
               w3_ref, p_ref):
    inv = _inv_deg(deg_ref)
    aq = (aggq_ref[0] + aggq_ref[1]) * inv
    u = jnp.dot(w2r_ref[...], w3_ref[...], preferred_element_type=jnp.float32)
    p = jnp.dot(h1a_ref[...], u[:_HW], preferred_element_type=jnp.float32)
    p = p + jnp.dot(h1b_ref[...], u[_HW:],
                    preferred_element_type=jnp.float32)
    c = jnp.dot(b2_ref[...][None, :], w3_ref[...],
                preferred_element_type=jnp.float32)
    p_ref[...] = p + aq + c


def _full(shape):
    nd = len(shape)
    return pl.BlockSpec(shape, lambda i: (0,) * nd)


_layer1 = pl.pallas_call(
    _layer1_tc,
    grid=(_N // _BR,),
    in_specs=[
        pl.BlockSpec((_BR, _DIN), lambda i: (i, 0)),
        pl.BlockSpec((_NC, _BR, _DH), lambda i: (0, i, 0)),
        pl.BlockSpec((_NC, _BR, _DEGW), lambda i: (0, i, 0)),
        _full((_DIN, _DH)),
        _full((_DIN, _DH)),
        _full((_DH,)),
        _full((_DH, _DH2)),
        _full((_DH2, _DOUT)),
    ],
    out_specs=[
        pl.BlockSpec((_BR, _HW), lambda i: (i, 0)),
        pl.BlockSpec((_BR, _HW), lambda i: (i, 0)),
        pl.BlockSpec((_BR, _DOUT), lambda i: (i, 0)),
    ],
    out_shape=[
        jax.ShapeDtypeStruct((_N, _HW), jnp.float32),
        jax.ShapeDtypeStruct((_N, _HW), jnp.float32),
        jax.ShapeDtypeStruct((_N, _DOUT), jnp.float32),
    ],
)

_layer2 = pl.pallas_call(
    _layer2_tc,
    grid=(_N // _BR,),
    in_specs=[
        pl.BlockSpec((_BR, _HW), lambda i: (i, 0)),
        pl.BlockSpec((_BR, _HW), lambda i: (i, 0)),
        pl.BlockSpec((_NC, _BR, _DOUT), lambda i: (0, i, 0)),
        pl.BlockSpec((_NC, _BR, _DEGW), lambda i: (0, i, 0)),
        _full((_DH, _DH2)),
        _full((_DH2,)),
        _full((_DH2, _DOUT)),
    ],
    out_specs=pl.BlockSpec((_BR, _DOUT), lambda i: (i, 0)),
    out_shape=jax.ShapeDtypeStruct((_N, _DOUT), jnp.float32),
)


def kernel(x, edge_index, edge_time, seed_idx, W1r, W1n, b1, W2r, W2n, b2,
           wt, bt, W3, b3):
    src = edge_index[0]
    dst = edge_index[1]
    pad = _EP - _E
    srcw = jnp.concatenate(
        [src, jnp.zeros((pad,), jnp.int32)]).reshape(_NW, _CH, _SB)
    dstw = jnp.concatenate(
        [dst, jnp.full((pad,), _N, jnp.int32)]).reshape(_NW, _CH, _SB)
    etw = jnp.concatenate(
        [edge_time, jnp.zeros((pad,), jnp.float32)]).reshape(_NW, _CH, _SB)
    seedw = seed_idx.reshape(_NW, _NSEED // _NW)
    wt16 = jnp.broadcast_to(wt.astype(jnp.float32), (16,))
    bt16 = jnp.broadcast_to(bt.astype(jnp.float32), (16,))

    aggx, degp = _make_segsum(_DH, True)(x, srcw, dstw)
    h1a, h1b, q = _layer1(x, aggx, degp, W1r, W1n, b1, W2n, W3)
    aggq = _make_segsum(_DOUT, False)(q, srcw, dstw)
    p = _layer2(h1a, h1b, aggq, degp, W2r, b2, W3)
    agg3 = _segsum_gate(p, srcw, dstw, etw, wt16, bt16)
    out = _final(agg3[0], agg3[1], degp[0], degp[1], b3, seedw)
    return out


# speedup vs baseline: 1.1780x; 1.0002x over previous
import functools

import jax
import jax.numpy as jnp
from jax import lax
from jax.experimental import pallas as pl
from jax.experimental.pallas import tpu as pltpu
from jax.experimental.pallas import tpu_sc as plsc

_N = 10000
_E = 320000
_DIN = 128
_DH = 128
_DH2 = 256
_DOUT = 32
_NSEED = 1024

_NC = 2
_NS = 16
_NW = _NC * _NS
_SB = 128
_CH = -(-_E // (_NW * _SB))
_EP = _NW * _SB * _CH
_NP = 10032
_RPT = _NP // _NS
_DEGW = 16
_HW = 64

_BR = 2000


def _mesh():
    return plsc.VectorSubcoreMesh(
        core_axis_name="c", subcore_axis_name="s",
        num_cores=_NC, num_subcores=_NS)


def _fill_rows(ref, nrows, ncol, value):
    v = jnp.full((16,), value, jnp.float32)

    def body(r, carry):
        for k in range(ncol // 16):
            ref[r, pl.ds(k * 16, 16)] = v
        return carry

    lax.fori_loop(0, nrows, body, 0)


def _segsum_body(feat, with_deg, *refs):
    if with_deg:
        (x_hbm, srcw, dstw, agg_hbm, deg_hbm,
         idx_v, dst_v, rows, ones_v, zdeg_v, acc_sh, accd_sh, sem) = refs
    else:
        (x_hbm, srcw, dstw, agg_hbm,
         idx_v, dst_v, rows, acc_sh, sem) = refs
    c = lax.axis_index("c")
    s = lax.axis_index("s")
    wid = c * _NS + s
    base = s * _RPT

    _fill_rows(rows, _SB, feat, 0.0)
    if with_deg:
        _fill_rows(ones_v, _SB, _DEGW, 1.0)
        _fill_rows(zdeg_v, _SB, _DEGW, 0.0)
    pltpu.sync_copy(srcw.at[wid], idx_v)
    pltpu.sync_copy(dstw.at[wid], dst_v)
    off = 0
    while off < _RPT:
        n = min(_SB, _RPT - off)
        pltpu.sync_copy(rows.at[pl.ds(0, n)],
                        acc_sh.at[pl.ds(base + off, n)])
        if with_deg:
            pltpu.sync_copy(zdeg_v.at[pl.ds(0, n)],
                            accd_sh.at[pl.ds(base + off, n)])
        off += n
    plsc.subcore_barrier()

    def body(j, carry):
        pltpu.async_copy(x_hbm.at[idx_v.at[j]], rows, sem).wait()
        pltpu.sync_copy(rows, acc_sh.at[dst_v.at[j]], add=True)
        if with_deg:
            pltpu.sync_copy(ones_v, accd_sh.at[dst_v.at[j]], add=True)
        return carry

    lax.fori_loop(0, _CH, body, 0)
    plsc.subcore_barrier()
    pltpu.sync_copy(acc_sh.at[pl.ds(base, _RPT)],
                    agg_hbm.at[c, pl.ds(base, _RPT)])
    if with_deg:
        pltpu.sync_copy(accd_sh.at[pl.ds(base, _RPT)],
                        deg_hbm.at[c, pl.ds(base, _RPT)])


def _make_segsum(feat, with_deg):
    out_type = [jax.ShapeDtypeStruct((_NC, _NP, feat), jnp.float32)]
    scratch = [
        pltpu.VMEM((_CH, _SB), jnp.int32),
        pltpu.VMEM((_CH, _SB), jnp.int32),
        pltpu.VMEM((_SB, feat), jnp.float32),
    ]
    if with_deg:
        out_type.append(jax.ShapeDtypeStruct((_NC, _NP, _DEGW), jnp.float32))
        scratch += [
            pltpu.VMEM((_SB, _DEGW), jnp.float32),
            pltpu.VMEM((_SB, _DEGW), jnp.float32),
        ]
    scratch.append(pltpu.VMEM_SHARED((_NP, feat), jnp.float32))
    if with_deg:
        scratch.append(pltpu.VMEM_SHARED((_NP, _DEGW), jnp.float32))
    scratch.append(pltpu.SemaphoreType.DMA)
    return pl.kernel(
        functools.partial(_segsum_body, feat, with_deg),
        out_type=out_type if with_deg else out_type[0],
        mesh=_mesh(),
        scratch_types=scratch,
        compiler_params=pltpu.CompilerParams(use_tc_tiling_on_sc=False),
    )


def _segsum_gate_body(p_hbm, srcw, dstw, etw, wt16, bt16, agg_hbm,
                      idx_v, dst_v, et_v, rows, wt_v, bt_v,
                      acc_sh, sem):
    c = lax.axis_index("c")
    s = lax.axis_index("s")
    wid = c * _NS + s
    base = s * _RPT

    _fill_rows(rows, _SB, _DOUT, 0.0)
    pltpu.sync_copy(srcw.at[wid], idx_v)
    pltpu.sync_copy(dstw.at[wid], dst_v)
    pltpu.sync_copy(etw.at[wid], et_v)
    pltpu.sync_copy(wt16, wt_v)
    pltpu.sync_copy(bt16, bt_v)
    off = 0
    while off < _RPT:
        n = min(_SB, _RPT - off)
        pltpu.sync_copy(rows.at[pl.ds(0, n)],
                        acc_sh.at[pl.ds(base + off, n)])
        off += n
    plsc.subcore_barrier()

    def body(j, carry):
        pltpu.async_copy(p_hbm.at[idx_v.at[j]], rows, sem).wait()
        wv = wt_v[...]
        bv = bt_v[...]
        for g in range(_SB // 16):
            z = et_v[j, pl.ds(g * 16, 16)]
            gval = 1.0 / (1.0 + jnp.exp(-(z * wv + bv)))
            for l in range(16):
                e = g * 16 + l
                gvec = jnp.full((16,), gval[l], jnp.float32)
                for k in range(_DOUT // 16):
                    sl = pl.ds(k * 16, 16)
                    rows[e, sl] = rows[e, sl] * gvec
        pltpu.sync_copy(rows, acc_sh.at[dst_v.at[j]], add=True)
        return carry

    lax.fori_loop(0, _CH, body, 0)
    plsc.subcore_barrier()
    pltpu.sync_copy(acc_sh.at[pl.ds(base, _RPT)],
                    agg_hbm.at[c, pl.ds(base, _RPT)])


_segsum_gate = pl.kernel(
    _segsum_gate_body,
    out_type=jax.ShapeDtypeStruct((_NC, _NP, _DOUT), jnp.float32),
    mesh=_mesh(),
    scratch_types=[
        pltpu.VMEM((_CH, _SB), jnp.int32),
        pltpu.VMEM((_CH, _SB), jnp.int32),
        pltpu.VMEM((_CH, _SB), jnp.float32),
        pltpu.VMEM((_SB, _DOUT), jnp.float32),
        pltpu.VMEM((16,), jnp.float32),
        pltpu.VMEM((16,), jnp.float32),
        pltpu.VMEM_SHARED((_NP, _DOUT), jnp.float32),
        pltpu.SemaphoreType.DMA,
    ],
    compiler_params=pltpu.CompilerParams(use_tc_tiling_on_sc=False),
)


def _final_body(a0_hbm, a1_hbm, d0_hbm, d1_hbm, b3_hbm, seedw, out_hbm,
                sidx_v, a0, a1, d0, d1, b3_v, outb, sem):
    c = lax.axis_index("c")
    s = lax.axis_index("s")
    wid = c * _NS + s
    spw = _NSEED // _NW

    pltpu.sync_copy(seedw.at[wid], sidx_v)
    pltpu.sync_copy(b3_hbm, b3_v)
    pltpu.async_copy(a0_hbm.at[sidx_v], a0, sem).wait()
    pltpu.async_copy(a1_hbm.at[sidx_v], a1, sem).wait()
    pltpu.async_copy(d0_hbm.at[sidx_v], d0, sem).wait()
    pltpu.async_copy(d1_hbm.at[sidx_v], d1, sem).wait()

    for r in range(spw):
        dvec = d0[r, pl.ds(0, 16)] + d1[r, pl.ds(0, 16)]
        invv = 1.0 / jnp.maximum(dvec, 1.0)
        iv = jnp.full((16,), invv[0], jnp.float32)
        for k in range(_DOUT // 16):
            sl = pl.ds(k * 16, 16)
            outb[r, sl] = (a0[r, sl] + a1[r, sl]) * iv + b3_v[sl]

    pltpu.sync_copy(outb, out_hbm.at[pl.ds(wid * spw, spw)])


_final = pl.kernel(
    _final_body,
    out_type=jax.ShapeDtypeStruct((_NSEED, _DOUT), jnp.float32),
    mesh=_mesh(),
    scratch_types=[
        pltpu.VMEM((_NSEED // _NW,), jnp.int32),
        pltpu.VMEM((_NSEED // _NW, _DOUT), jnp.float32),
        pltpu.VMEM((_NSEED // _NW, _DOUT), jnp.float32),
        pltpu.VMEM((_NSEED // _NW, _DEGW), jnp.float32),
        pltpu.VMEM((_NSEED // _NW, _DEGW), jnp.float32),
        pltpu.VMEM((_DOUT,), jnp.float32),
        pltpu.VMEM((_NSEED // _NW, _DOUT), jnp.float32),
        pltpu.SemaphoreType.DMA,
    ],
    compiler_params=pltpu.CompilerParams(use_tc_tiling_on_sc=False),
)


def _inv_deg(deg_ref):
    deg = deg_ref[0, :, 0:1] + deg_ref[1, :, 0:1]
    return 1.0 / jnp.maximum(deg, 1.0)


def _layer1_tc(x_ref, agg_ref, deg_ref, w1r_ref, w1n_ref,
               b1_ref, w2n_ref, w3_ref, h1a_ref, h1b_ref, q_ref):
    inv = _inv_deg(deg_ref)
    a = (agg_ref[0] + agg_ref[1]) * inv
    h = jnp.dot(x_ref[...], w1r_ref[...], preferred_element_type=jnp.float32)
    h = h + jnp.dot(a, w1n_ref[...], preferred_element_type=jnp.float32)
    h = jax.nn.gelu(h + b1_ref[...][None, :])
    h1a_ref[...] = h[:, :_HW]
    h1b_ref[...] = h[:, _HW:]
    v = jnp.dot(w2n_ref[...], w3_ref[...], preferred_element_type=jnp.float32)
    q_ref[...] = jnp.dot(h, v, preferred_element_type=jnp.float32)


def _layer2_tc(h1a_ref, h1b_ref, aggq_ref, deg_ref, w2r_ref, b2_ref,
               w3_ref, p_ref):
    inv = _inv_deg(deg_ref)
    aq = (aggq_ref[0] + aggq_ref[1]) * inv
    u = jnp.dot(w2r_ref[...], w3_ref[...], preferred_element_type=jnp.float32)
    p = jnp.dot(h1a_ref[...], u[:_HW], preferred_element_type=jnp.float32)
    p = p + jnp.dot(h1b_ref[...], u[_HW:],
                    preferred_element_type=jnp.float32)
    c = jnp.dot(b2_ref[...][None, :], w3_ref[...],
                preferred_element_type=jnp.float32)
    p_ref[...] = p + aq + c


def _full(shape):
    nd = len(shape)
    return pl.BlockSpec(shape, lambda i: (0,) * nd)


_layer1 = pl.pallas_call(
    _layer1_tc,
    grid=(_N // _BR,),
    in_specs=[
        pl.BlockSpec((_BR, _DIN), lambda i: (i, 0)),
        pl.BlockSpec((_NC, _BR, _DH), lambda i: (0, i, 0)),
        pl.BlockSpec((_NC, _BR, _DEGW), lambda i: (0, i, 0)),
        _full((_DIN, _DH)),
        _full((_DIN, _DH)),
        _full((_DH,)),
        _full((_DH, _DH2)),
        _full((_DH2, _DOUT)),
    ],
    out_specs=[
        pl.BlockSpec((_BR, _HW), lambda i: (i, 0)),
        pl.BlockSpec((_BR, _HW), lambda i: (i, 0)),
        pl.BlockSpec((_BR, _DOUT), lambda i: (i, 0)),
    ],
    out_shape=[
        jax.ShapeDtypeStruct((_N, _HW), jnp.float32),
        jax.ShapeDtypeStruct((_N, _HW), jnp.float32),
        jax.ShapeDtypeStruct((_N, _DOUT), jnp.float32),
    ],
)

_layer2 = pl.pallas_call(
    _layer2_tc,
    grid=(_N // _BR,),
    in_specs=[
        pl.BlockSpec((_BR, _HW), lambda i: (i, 0)),
        pl.BlockSpec((_BR, _HW), lambda i: (i, 0)),
        pl.BlockSpec((_NC, _BR, _DOUT), lambda i: (0, i, 0)),
        pl.BlockSpec((_NC, _BR, _DEGW), lambda i: (0, i, 0)),
        _full((_DH, _DH2)),
        _full((_DH2,)),
        _full((_DH2, _DOUT)),
    ],
    out_specs=pl.BlockSpec((_BR, _DOUT), lambda i: (i, 0)),
    out_shape=jax.ShapeDtypeStruct((_N, _DOUT), jnp.float32),
)


def kernel(x, edge_index, edge_time, seed_idx, W1r, W1n, b1, W2r, W2n, b2,
           wt, bt, W3, b3):
    src = edge_index[0]
    dst = edge_index[1]
    pad = _EP - _E
    srcw = jnp.concatenate(
        [src, jnp.zeros((pad,), jnp.int32)]).reshape(_NW, _CH, _SB)
    dump = _N + (jnp.arange(pad, dtype=jnp.int32) % (_NP - _N))
    dstw = jnp.concatenate([dst, dump]).reshape(_NW, _CH, _SB)
    etw = jnp.concatenate(
        [edge_time, jnp.zeros((pad,), jnp.float32)]).reshape(_NW, _CH, _SB)
    seedw = seed_idx.reshape(_NW, _NSEED // _NW)
    wt16 = jnp.broadcast_to(wt.astype(jnp.float32), (16,))
    bt16 = jnp.broadcast_to(bt.astype(jnp.float32), (16,))

    aggx, degp = _make_segsum(_DH, True)(x, srcw, dstw)
    h1a, h1b, q = _layer1(x, aggx, degp, W1r, W1n, b1, W2n, W3)
    aggq = _make_segsum(_DOUT, False)(q, srcw, dstw)
    p = _layer2(h1a, h1b, aggq, degp, W2r, b2, W3)
    agg3 = _segsum_gate(p, srcw, dstw, etw, wt16, bt16)
    out = _final(agg3[0], agg3[1], degp[0], degp[1], b3, seedw)
    return out
